# baseline (speedup 1.0000x reference)
import functools

import jax
import jax.numpy as jnp
from jax import lax
from jax.experimental import pallas as pl
from jax.experimental.pallas import tpu as pltpu
from jax.experimental.pallas import tpu_sc as plsc

_B = 16384
_D = 128


def _build_sc_gather(n):
    info = plsc.get_sparse_core_info()
    nc, ns = info.num_cores, info.num_subcores
    nw = nc * ns
    b_per_w = n // nw
    ch = 128
    nq = b_per_w // ch
    mesh = plsc.VectorSubcoreMesh(core_axis_name="c", subcore_axis_name="s")

    @functools.partial(
        pl.kernel,
        out_type=jax.ShapeDtypeStruct((n, _D), jnp.int32),
        mesh=mesh,
        scratch_types=[
            pltpu.VMEM((b_per_w,), jnp.int32),
            pltpu.VMEM((b_per_w,), jnp.int32),
            pltpu.VMEM((2, ch, _D), jnp.float32),
            pltpu.VMEM((2, ch, _D), jnp.float32),
            pltpu.VMEM((1, ch, _D), jnp.int32),
            pltpu.SemaphoreType.DMA,
            pltpu.SemaphoreType.DMA,
            pltpu.SemaphoreType.DMA,
            pltpu.SemaphoreType.DMA,
            pltpu.SemaphoreType.DMA,
        ],
    )
    def gather_k(uids_hbm, gids_hbm, uemb_hbm, iemb_hbm, pk_out,
                 idxu_v, idxg_v, fu, fi, pk,
                 sem_gu0, sem_gu1, sem_gi0, sem_gi1, sem_w):
        sem_gu = (sem_gu0, sem_gu1)
        sem_gi = (sem_gi0, sem_gi1)
        wid = lax.axis_index("s") * nc + lax.axis_index("c")
        base = wid * b_per_w
        pltpu.sync_copy(uids_hbm.at[pl.ds(base, b_per_w)], idxu_v)
        pltpu.sync_copy(gids_hbm.at[pl.ds(base, b_per_w)], idxg_v)

        def bf16_round(x):
            return lax.bitcast_convert_type(x, jnp.int32) + 0x8000

        def convert(b):
            def body(i, carry):
                for g in range(0, _D, 16):
                    au = lax.shift_right_logical(
                        bf16_round(fu[b, i, pl.ds(g, 16)]), 16)
                    ai = bf16_round(fi[b, i, pl.ds(g, 16)]) & jnp.int32(
                        -65536)
                    pk[0, i, pl.ds(g, 16)] = au | ai
                return carry
            lax.fori_loop(0, ch, body, 0)

        gcp = {}

        def start_gather(q):
            b = q % 2
            cu = pltpu.make_async_copy(
                uemb_hbm.at[idxu_v.at[pl.ds(q * ch, ch)]], fu.at[b],
                sem_gu[b])
            ci = pltpu.make_async_copy(
                iemb_hbm.at[idxg_v.at[pl.ds(q * ch, ch)]], fi.at[b],
                sem_gi[b])
            cu.start()
            ci.start()
            gcp[q] = (cu, ci)

        start_gather(0)
        wbs = {}
        for q in range(nq):
            b = q % 2
            if q + 1 < nq:
                start_gather(q + 1)
            for c in gcp.pop(q):
                c.wait()
            if q >= 1:
                wbs.pop(q - 1).wait()
            convert(b)
            w = pltpu.make_async_copy(
                pk.at[0], pk_out.at[pl.ds(base + q * ch, ch)], sem_w)
            w.start()
            wbs[q] = w
        wbs.pop(nq - 1).wait()

    return gather_k


_sc_gather = _build_sc_gather(_B)

_BLK = 4096


def _tc_mlp_dot(pk_rows, W1, b1, W2, b2):
    n = pk_rows.shape[0]
    nblk = n // _BLK

    def body(pk_ref, w1_ref, b1_ref, w2_ref, b2_ref, out_ref):
        w = pk_ref[...]
        u = lax.bitcast_convert_type(lax.shift_left(w, 16), jnp.float32)
        it = lax.bitcast_convert_type(w & jnp.int32(-65536), jnp.float32)
        h = jnp.dot(it, w1_ref[...], preferred_element_type=jnp.float32)
        h = jnp.maximum(h + b1_ref[...], 0.0)
        h = jnp.dot(h, w2_ref[...], preferred_element_type=jnp.float32)
        h = h + b2_ref[...]
        p = u * h
        out_ref[...] = jnp.sum(p.T, axis=0)[None, None, :]

    out = pl.pallas_call(
        body,
        grid=(nblk,),
        in_specs=[
            pl.BlockSpec((_BLK, _D), lambda i: (i, 0)),
            pl.BlockSpec((_D, _D), lambda i: (0, 0)),
            pl.BlockSpec((1, _D), lambda i: (0, 0)),
            pl.BlockSpec((_D, _D), lambda i: (0, 0)),
            pl.BlockSpec((1, _D), lambda i: (0, 0)),
        ],
        out_specs=pl.BlockSpec((1, 1, _BLK), lambda i: (i, 0, 0)),
        out_shape=jax.ShapeDtypeStruct((nblk, 1, _BLK), jnp.float32),
    )(pk_rows, W1, b1.reshape(1, _D), W2, b2.reshape(1, _D))
    return out.reshape(n)


def kernel(uids, gids, user_emb, item_emb, W1, b1, W2, b2):
    uids = uids.astype(jnp.int32)
    gids = gids.astype(jnp.int32)
    pk_rows = _sc_gather(uids, gids, user_emb, item_emb)
    return _tc_mlp_dot(pk_rows, W1, b1, W2, b2)


# speedup vs baseline: 1.0240x; 1.0240x over previous
import functools

import jax
import jax.numpy as jnp
from jax import lax
from jax.experimental import pallas as pl
from jax.experimental.pallas import tpu as pltpu
from jax.experimental.pallas import tpu_sc as plsc

_B = 16384
_D = 128


def _build_sc_gather(n):
    info = plsc.get_sparse_core_info()
    nc, ns = info.num_cores, info.num_subcores
    nw = nc * ns
    b_per_w = n // nw
    ch = 128
    nq = b_per_w // ch
    mesh = plsc.VectorSubcoreMesh(core_axis_name="c", subcore_axis_name="s")

    @functools.partial(
        pl.kernel,
        out_type=jax.ShapeDtypeStruct((n, _D), jnp.int32),
        mesh=mesh,
        scratch_types=[
            pltpu.VMEM((b_per_w,), jnp.int32),
            pltpu.VMEM((b_per_w,), jnp.int32),
            pltpu.VMEM((2, ch, _D), jnp.float32),
            pltpu.VMEM((2, ch, _D), jnp.float32),
            pltpu.VMEM((1, ch, _D), jnp.int32),
            pltpu.SemaphoreType.DMA,
            pltpu.SemaphoreType.DMA,
            pltpu.SemaphoreType.DMA,
            pltpu.SemaphoreType.DMA,
            pltpu.SemaphoreType.DMA,
        ],
    )
    def gather_k(uids_hbm, gids_hbm, uemb_hbm, iemb_hbm, pk_out,
                 idxu_v, idxg_v, fu, fi, pk,
                 sem_gu0, sem_gu1, sem_gi0, sem_gi1, sem_w):
        sem_gu = (sem_gu0, sem_gu1)
        sem_gi = (sem_gi0, sem_gi1)
        wid = lax.axis_index("s") * nc + lax.axis_index("c")
        base = wid * b_per_w
        pltpu.sync_copy(uids_hbm.at[pl.ds(base, b_per_w)], idxu_v)
        pltpu.sync_copy(gids_hbm.at[pl.ds(base, b_per_w)], idxg_v)

        def bf16_round(x):
            return lax.bitcast_convert_type(x, jnp.int32) + 0x8000

        def convert(b):
            def body(i, carry):
                for g in range(0, _D, 16):
                    au = lax.shift_right_logical(
                        bf16_round(fu[b, i, pl.ds(g, 16)]), 16)
                    ai = bf16_round(fi[b, i, pl.ds(g, 16)]) & jnp.int32(
                        -65536)
                    pk[0, i, pl.ds(g, 16)] = au | ai
                return carry
            lax.fori_loop(0, ch, body, 0)

        gcp = {}

        def start_gather(q):
            b = q % 2
            cu = pltpu.make_async_copy(
                uemb_hbm.at[idxu_v.at[pl.ds(q * ch, ch)]], fu.at[b],
                sem_gu[b])
            ci = pltpu.make_async_copy(
                iemb_hbm.at[idxg_v.at[pl.ds(q * ch, ch)]], fi.at[b],
                sem_gi[b])
            cu.start()
            ci.start()
            gcp[q] = (cu, ci)

        start_gather(0)
        wbs = {}
        for q in range(nq):
            b = q % 2
            for c in gcp.pop(q):
                c.wait()
            if q + 1 < nq:
                start_gather(q + 1)
            if q >= 1:
                wbs.pop(q - 1).wait()
            convert(b)
            w = pltpu.make_async_copy(
                pk.at[0], pk_out.at[pl.ds(base + q * ch, ch)], sem_w)
            w.start()
            wbs[q] = w
        wbs.pop(nq - 1).wait()

    return gather_k


_sc_gather = _build_sc_gather(_B)

_BLK = 4096


def _tc_mlp_dot(pk_rows, W1, b1, W2, b2):
    n = pk_rows.shape[0]
    nblk = n // _BLK

    def body(pk_ref, w1_ref, b1_ref, w2_ref, b2_ref, out_ref):
        w = pk_ref[...]
        u = lax.bitcast_convert_type(lax.shift_left(w, 16), jnp.float32)
        it = lax.bitcast_convert_type(w & jnp.int32(-65536), jnp.float32)
        h = jnp.dot(it, w1_ref[...], preferred_element_type=jnp.float32)
        h = jnp.maximum(h + b1_ref[...], 0.0)
        h = jnp.dot(h, w2_ref[...], preferred_element_type=jnp.float32)
        h = h + b2_ref[...]
        p = u * h
        out_ref[...] = jnp.sum(p.T, axis=0)[None, None, :]

    out = pl.pallas_call(
        body,
        grid=(nblk,),
        in_specs=[
            pl.BlockSpec((_BLK, _D), lambda i: (i, 0)),
            pl.BlockSpec((_D, _D), lambda i: (0, 0)),
            pl.BlockSpec((1, _D), lambda i: (0, 0)),
            pl.BlockSpec((_D, _D), lambda i: (0, 0)),
            pl.BlockSpec((1, _D), lambda i: (0, 0)),
        ],
        out_specs=pl.BlockSpec((1, 1, _BLK), lambda i: (i, 0, 0)),
        out_shape=jax.ShapeDtypeStruct((nblk, 1, _BLK), jnp.float32),
    )(pk_rows, W1, b1.reshape(1, _D), W2, b2.reshape(1, _D))
    return out.reshape(n)


def kernel(uids, gids, user_emb, item_emb, W1, b1, W2, b2):
    uids = uids.astype(jnp.int32)
    gids = gids.astype(jnp.int32)
    pk_rows = _sc_gather(uids, gids, user_emb, item_emb)
    return _tc_mlp_dot(pk_rows, W1, b1, W2, b2)


# speedup vs baseline: 1.0354x; 1.0111x over previous
import functools

import jax
import jax.numpy as jnp
from jax import lax
from jax.experimental import pallas as pl
from jax.experimental.pallas import tpu as pltpu
from jax.experimental.pallas import tpu_sc as plsc

_B = 16384
_D = 128


def _build_sc_gather(n):
    info = plsc.get_sparse_core_info()
    nc, ns = info.num_cores, info.num_subcores
    nw = nc * ns
    b_per_w = n // nw
    ch = 128
    nq = b_per_w // ch
    mesh = plsc.VectorSubcoreMesh(core_axis_name="c", subcore_axis_name="s")

    @functools.partial(
        pl.kernel,
        out_type=jax.ShapeDtypeStruct((n, _D), jnp.int32),
        mesh=mesh,
        scratch_types=[
            pltpu.VMEM((b_per_w,), jnp.int32),
            pltpu.VMEM((b_per_w,), jnp.int32),
            pltpu.VMEM((2, ch, _D), jnp.float32),
            pltpu.VMEM((2, ch, _D), jnp.float32),
            pltpu.VMEM((1, ch, _D), jnp.int32),
            pltpu.SemaphoreType.DMA,
            pltpu.SemaphoreType.DMA,
            pltpu.SemaphoreType.DMA,
            pltpu.SemaphoreType.DMA,
            pltpu.SemaphoreType.DMA,
        ],
    )
    def gather_k(uids_hbm, gids_hbm, uemb_hbm, iemb_hbm, pk_out,
                 idxu_v, idxg_v, fu, fi, pk,
                 sem_gu0, sem_gu1, sem_gi0, sem_gi1, sem_w):
        sem_gu = (sem_gu0, sem_gu1)
        sem_gi = (sem_gi0, sem_gi1)
        wid = lax.axis_index("s") * nc + lax.axis_index("c")
        base = wid * b_per_w
        c1 = pltpu.make_async_copy(
            uids_hbm.at[pl.ds(base, b_per_w)], idxu_v, sem_w)
        c2 = pltpu.make_async_copy(
            gids_hbm.at[pl.ds(base, b_per_w)], idxg_v, sem_w)
        c1.start()
        c2.start()
        c1.wait()
        c2.wait()

        def bf16_round(x):
            return lax.bitcast_convert_type(x, jnp.int32) + 0x8000

        def convert(b):
            def body(i, carry):
                for g in range(0, _D, 16):
                    au = lax.shift_right_logical(
                        bf16_round(fu[b, i, pl.ds(g, 16)]), 16)
                    ai = bf16_round(fi[b, i, pl.ds(g, 16)]) & jnp.int32(
                        -65536)
                    pk[0, i, pl.ds(g, 16)] = au | ai
                return carry
            lax.fori_loop(0, ch, body, 0)

        gcp = {}

        def start_gather(q):
            b = q % 2
            cu = pltpu.make_async_copy(
                uemb_hbm.at[idxu_v.at[pl.ds(q * ch, ch)]], fu.at[b],
                sem_gu[b])
            ci = pltpu.make_async_copy(
                iemb_hbm.at[idxg_v.at[pl.ds(q * ch, ch)]], fi.at[b],
                sem_gi[b])
            cu.start()
            ci.start()
            gcp[q] = (cu, ci)

        start_gather(0)
        wbs = {}
        for q in range(nq):
            b = q % 2
            for c in gcp.pop(q):
                c.wait()
            if q + 1 < nq:
                start_gather(q + 1)
            if q >= 1:
                wbs.pop(q - 1).wait()
            convert(b)
            w = pltpu.make_async_copy(
                pk.at[0], pk_out.at[pl.ds(base + q * ch, ch)], sem_w)
            w.start()
            wbs[q] = w
        wbs.pop(nq - 1).wait()

    return gather_k


_sc_gather = _build_sc_gather(_B)

_BLK = 4096


def _tc_mlp_dot(pk_rows, W1, b1, W2, b2):
    n = pk_rows.shape[0]
    nblk = n // _BLK

    def body(pk_ref, w1_ref, b1_ref, w2_ref, b2_ref, out_ref):
        w = pk_ref[...]
        u = lax.bitcast_convert_type(lax.shift_left(w, 16), jnp.float32)
        it = lax.bitcast_convert_type(w & jnp.int32(-65536), jnp.float32)
        h = jnp.dot(it, w1_ref[...], preferred_element_type=jnp.float32)
        h = jnp.maximum(h + b1_ref[...], 0.0)
        h = jnp.dot(h, w2_ref[...], preferred_element_type=jnp.float32)
        h = h + b2_ref[...]
        p = u * h
        out_ref[...] = jnp.sum(p.T, axis=0)[None, None, :]

    out = pl.pallas_call(
        body,
        grid=(nblk,),
        in_specs=[
            pl.BlockSpec((_BLK, _D), lambda i: (i, 0)),
            pl.BlockSpec((_D, _D), lambda i: (0, 0)),
            pl.BlockSpec((1, _D), lambda i: (0, 0)),
            pl.BlockSpec((_D, _D), lambda i: (0, 0)),
            pl.BlockSpec((1, _D), lambda i: (0, 0)),
        ],
        out_specs=pl.BlockSpec((1, 1, _BLK), lambda i: (i, 0, 0)),
        out_shape=jax.ShapeDtypeStruct((nblk, 1, _BLK), jnp.float32),
    )(pk_rows, W1, b1.reshape(1, _D), W2, b2.reshape(1, _D))
    return out.reshape(n)


def kernel(uids, gids, user_emb, item_emb, W1, b1, W2, b2):
    uids = uids.astype(jnp.int32)
    gids = gids.astype(jnp.int32)
    pk_rows = _sc_gather(uids, gids, user_emb, item_emb)
    return _tc_mlp_dot(pk_rows, W1, b1, W2, b2)


# speedup vs baseline: 1.0615x; 1.0252x over previous
import functools

import jax
import jax.numpy as jnp
from jax import lax
from jax.experimental import pallas as pl
from jax.experimental.pallas import tpu as pltpu
from jax.experimental.pallas import tpu_sc as plsc

_B = 16384
_D = 128


def _build_sc_gather(n):
    info = plsc.get_sparse_core_info()
    nc, ns = info.num_cores, info.num_subcores
    nw = nc * ns
    b_per_w = n // nw
    ch = 128
    nq = b_per_w // ch
    mesh = plsc.VectorSubcoreMesh(core_axis_name="c", subcore_axis_name="s")

    @functools.partial(
        pl.kernel,
        out_type=jax.ShapeDtypeStruct((n, _D), jnp.int32),
        mesh=mesh,
        scratch_types=[
            pltpu.VMEM((b_per_w,), jnp.int32),
            pltpu.VMEM((b_per_w,), jnp.int32),
            pltpu.VMEM((2, ch, _D), jnp.float32),
            pltpu.VMEM((2, ch, _D), jnp.float32),
            pltpu.VMEM((1, ch, _D), jnp.int32),
            pltpu.SemaphoreType.DMA,
            pltpu.SemaphoreType.DMA,
            pltpu.SemaphoreType.DMA,
            pltpu.SemaphoreType.DMA,
            pltpu.SemaphoreType.DMA,
        ],
    )
    def gather_k(uids_hbm, gids_hbm, uemb_hbm, iemb_hbm, pk_out,
                 idxu_v, idxg_v, fu, fi, pk,
                 sem_gu0, sem_gu1, sem_gi0, sem_gi1, sem_w):
        sem_gu = (sem_gu0, sem_gu1)
        sem_gi = (sem_gi0, sem_gi1)
        wid = lax.axis_index("s") * nc + lax.axis_index("c")
        base = wid * b_per_w
        c1 = pltpu.make_async_copy(
            uids_hbm.at[pl.ds(base, b_per_w)], idxu_v, sem_w)
        c2 = pltpu.make_async_copy(
            gids_hbm.at[pl.ds(base, b_per_w)], idxg_v, sem_w)
        c1.start()
        c2.start()
        c1.wait()
        c2.wait()

        def bf16_round(x):
            return lax.bitcast_convert_type(x, jnp.int32) + 0x8000

        def convert(b):
            def body(i, carry):
                for g in range(0, _D, 16):
                    au = lax.shift_right_logical(
                        bf16_round(fu[b, i, pl.ds(g, 16)]), 16)
                    ai = bf16_round(fi[b, i, pl.ds(g, 16)]) & jnp.int32(
                        -65536)
                    pk[0, i, pl.ds(g, 16)] = au | ai
                return carry
            lax.fori_loop(0, ch, body, 0)

        gcp = {}

        def start_gather(q):
            b = q % 2
            cu = pltpu.make_async_copy(
                uemb_hbm.at[idxu_v.at[pl.ds(q * ch, ch)]], fu.at[b],
                sem_gu[b])
            ci = pltpu.make_async_copy(
                iemb_hbm.at[idxg_v.at[pl.ds(q * ch, ch)]], fi.at[b],
                sem_gi[b])
            cu.start()
            ci.start()
            gcp[q] = (cu, ci)

        start_gather(0)
        wbs = {}
        for q in range(nq):
            b = q % 2
            for c in gcp.pop(q):
                c.wait()
            if q + 1 < nq:
                start_gather(q + 1)
            if q >= 1:
                wbs.pop(q - 1).wait()
            convert(b)
            w = pltpu.make_async_copy(
                pk.at[0], pk_out.at[pl.ds(base + q * ch, ch)], sem_w)
            w.start()
            wbs[q] = w
        wbs.pop(nq - 1).wait()

    return gather_k


_sc_gather = _build_sc_gather(_B)

_BLK = 8192


def _tc_mlp_dot(pk_rows, W1, b1, W2, b2):
    n = pk_rows.shape[0]
    nblk = n // _BLK

    def body(pk_ref, w1_ref, b1_ref, w2_ref, b2_ref, out_ref):
        w = pk_ref[...]
        u = lax.bitcast_convert_type(lax.shift_left(w, 16), jnp.float32)
        it = lax.bitcast_convert_type(w & jnp.int32(-65536), jnp.float32)
        h = jnp.dot(it, w1_ref[...], preferred_element_type=jnp.float32)
        h = jnp.maximum(h + b1_ref[...], 0.0)
        h = jnp.dot(h, w2_ref[...], preferred_element_type=jnp.float32)
        h = h + b2_ref[...]
        p = u * h
        out_ref[...] = jnp.sum(p.T, axis=0)[None, None, :]

    out = pl.pallas_call(
        body,
        grid=(nblk,),
        in_specs=[
            pl.BlockSpec((_BLK, _D), lambda i: (i, 0)),
            pl.BlockSpec((_D, _D), lambda i: (0, 0)),
            pl.BlockSpec((1, _D), lambda i: (0, 0)),
            pl.BlockSpec((_D, _D), lambda i: (0, 0)),
            pl.BlockSpec((1, _D), lambda i: (0, 0)),
        ],
        out_specs=pl.BlockSpec((1, 1, _BLK), lambda i: (i, 0, 0)),
        out_shape=jax.ShapeDtypeStruct((nblk, 1, _BLK), jnp.float32),
    )(pk_rows, W1, b1.reshape(1, _D), W2, b2.reshape(1, _D))
    return out.reshape(n)


def kernel(uids, gids, user_emb, item_emb, W1, b1, W2, b2):
    uids = uids.astype(jnp.int32)
    gids = gids.astype(jnp.int32)
    pk_rows = _sc_gather(uids, gids, user_emb, item_emb)
    return _tc_mlp_dot(pk_rows, W1, b1, W2, b2)
